# baseline (speedup 1.0000x reference)
import functools

import jax
import jax.numpy as jnp
from jax import lax
from jax.experimental import pallas as pl
from jax.experimental.pallas import tpu as pltpu
from jax.experimental.pallas import tpu_sc as plsc

N = 10000
E = 320000
D = 128

NC = 2
NS = 16
NW = NC * NS
GW = 128
NQ = 2
EQ = E // NQ
EPW = EQ // NW
SCH = 40
SNCH = EPW // SCH
NP = 10240
RPT = NP // NS
ZR = 128

_PREC = lax.Precision.DEFAULT


def _dot(a, b):
    return lax.dot_general(a, b, (((1,), (0,)), ((), ())), precision=_PREC,
                           preferred_element_type=jnp.float32)


def _project_body(nf_ref, w0s_ref, w0r_ref, ps_ref, pr_ref):
    nf = nf_ref[...]
    ps_ref[...] = _dot(nf, w0s_ref[...])
    pr_ref[...] = _dot(nf, w0r_ref[...])


def _project(nf, w0s, w0r, bn=2000):
    grid = (N // bn,)
    return pl.pallas_call(
        _project_body,
        grid=grid,
        in_specs=[
            pl.BlockSpec((bn, D), lambda i: (i, 0)),
            pl.BlockSpec((D, D), lambda i: (0, 0)),
            pl.BlockSpec((D, D), lambda i: (0, 0)),
        ],
        out_specs=[
            pl.BlockSpec((bn, D), lambda i: (i, 0)),
            pl.BlockSpec((bn, D), lambda i: (i, 0)),
        ],
        out_shape=[
            jax.ShapeDtypeStruct((N, D), jnp.float32),
            jax.ShapeDtypeStruct((N, D), jnp.float32),
        ],
    )(nf, w0s, w0r)


def _sc_gather(ps, pr, senders, receivers):
    ne = senders.shape[0]
    mesh = plsc.VectorSubcoreMesh(core_axis_name="core",
                                  subcore_axis_name="subcore")

    @functools.partial(
        pl.kernel,
        out_type=(
            jax.ShapeDtypeStruct((ne, D), jnp.float32),
            jax.ShapeDtypeStruct((ne, D), jnp.float32),
        ),
        mesh=mesh,
        scratch_types=[
            pltpu.SemaphoreType.DMA,
            pltpu.SemaphoreType.DMA,
        ],
    )
    def gk(ps_hbm, pr_hbm, s_hbm, r_hbm, gs_hbm, gr_hbm, sem_s, sem_r):
        def body(si_v, ri_v, gs_v, gr_v):
            cs = pltpu.make_async_copy(ps_hbm.at[si_v.at[0]], gs_v, sem_s)
            cr = pltpu.make_async_copy(pr_hbm.at[ri_v.at[0]], gr_v, sem_r)
            cs.start()
            cr.start()
            cs.wait()
            cr.wait()

        pltpu.emit_pipeline(
            body,
            grid=(ne // GW,),
            in_specs=[
                pl.BlockSpec((1, GW), lambda i: (0, i)),
                pl.BlockSpec((1, GW), lambda i: (0, i)),
            ],
            out_specs=[
                pl.BlockSpec((GW, D), lambda i: (i, 0)),
                pl.BlockSpec((GW, D), lambda i: (i, 0)),
            ],
            core_axis_name=("core", "subcore"),
            dimension_semantics=(pltpu.PARALLEL,),
        )(s_hbm, r_hbm, gs_hbm, gr_hbm)

    return gk(ps, pr, senders.reshape(1, ne), receivers.reshape(1, ne))


def _edge_body(gs_ref, gr_ref, ef_ref, chain_ref, w0e_ref, b0_ref, w1_ref,
               b1_ref, g_ref, beta_ref, y_ref, out_ref):
    del chain_ref
    ef = ef_ref[...]
    x = gs_ref[...] + gr_ref[...] + _dot(ef, w0e_ref[...]) + b0_ref[...]
    h = jnp.maximum(x, 0.0)
    y = _dot(h, w1_ref[...]) + b1_ref[...]
    mu = jnp.mean(y, axis=1, keepdims=True)
    d = y - mu
    var = jnp.mean(d * d, axis=1, keepdims=True)
    yln = d * lax.rsqrt(var + 1e-5) * g_ref[...] + beta_ref[...]
    y_ref[...] = yln
    out_ref[...] = yln + ef


def _edge_mlp(gs, gr, ef, chain, q, w0e, b0, w1, b1, g, beta, be=2000):
    nb = EQ // be
    row = lambda i: (i, 0)
    rowq = lambda i: (i + q * nb, 0)
    full = lambda i: (0, 0)
    return pl.pallas_call(
        _edge_body,
        grid=(nb,),
        in_specs=[
            pl.BlockSpec((be, D), row),
            pl.BlockSpec((be, D), row),
            pl.BlockSpec((be, D), rowq),
            pl.BlockSpec(memory_space=pl.ANY),
            pl.BlockSpec((D, D), full),
            pl.BlockSpec((1, D), full),
            pl.BlockSpec((D, D), full),
            pl.BlockSpec((1, D), full),
            pl.BlockSpec((1, D), full),
            pl.BlockSpec((1, D), full),
        ],
        out_specs=[
            pl.BlockSpec((be, D), row),
            pl.BlockSpec((be, D), rowq),
        ],
        out_shape=[
            jax.ShapeDtypeStruct((EQ, D), jnp.float32),
            jax.ShapeDtypeStruct((E, D), jnp.float32),
        ],
        input_output_aliases={3: 1},
    )(gs, gr, ef, chain, w0e, b0, w1, b1, g, beta)


def _alloc_edge_buf():
    def body(o_ref):
        o_ref[...] = jnp.zeros((8, D), jnp.float32)

    return pl.pallas_call(
        body,
        grid=(1,),
        out_specs=pl.BlockSpec((8, D), lambda i: (0, 0)),
        out_shape=jax.ShapeDtypeStruct((E, D), jnp.float32),
    )()


def _sc_scatter(y, receivers):
    mesh = plsc.VectorSubcoreMesh(core_axis_name="core",
                                  subcore_axis_name="subcore")

    @functools.partial(
        pl.kernel,
        out_type=jax.ShapeDtypeStruct((NC, NP, D), jnp.float32),
        mesh=mesh,
        scratch_types=[
            pltpu.VMEM((SCH,), jnp.int32),
            pltpu.VMEM((SCH,), jnp.int32),
            pltpu.VMEM((SCH, D), jnp.float32),
            pltpu.VMEM((SCH, D), jnp.float32),
            pltpu.VMEM((ZR, D), jnp.float32),
            pltpu.VMEM_SHARED((NP, D), jnp.float32),
            pltpu.SemaphoreType.DMA,
            pltpu.SemaphoreType.DMA,
        ],
    )
    def sk(y_hbm, r_hbm, out_hbm, idx0_v, idx1_v, rows0_v, rows1_v, zbuf_v,
           acc_sh, sem0, sem1):
        cid = lax.axis_index("core")
        sid = lax.axis_index("subcore")
        wid = cid * NS + sid

        def start(c, idx_v, rows_v, sem):
            base = wid * EPW + c * SCH
            ci = pltpu.make_async_copy(r_hbm.at[pl.ds(base, SCH)], idx_v, sem)
            cy = pltpu.make_async_copy(y_hbm.at[pl.ds(base, SCH)], rows_v, sem)
            ci.start()
            cy.start()
            return ci, cy

        def drain(c, idx_v, rows_v, sem):
            ci = pltpu.make_async_copy(r_hbm.at[pl.ds(0, SCH)], idx_v, sem)
            cy = pltpu.make_async_copy(y_hbm.at[pl.ds(0, SCH)], rows_v, sem)
            ci.wait()
            cy.wait()

        @pl.loop(0, ZR)
        def _(r):
            @pl.loop(0, D // 16)
            def _(c):
                zbuf_v[r, pl.ds(c * 16, 16)] = jnp.zeros((16,), jnp.float32)

        @pl.loop(0, RPT // ZR)
        def _(j):
            pltpu.sync_copy(zbuf_v, acc_sh.at[pl.ds(sid * RPT + j * ZR, ZR)])

        plsc.subcore_barrier()

        start(0, idx0_v, rows0_v, sem0)

        @pl.loop(0, (SNCH - 1) // 2)
        def _(k):
            c0 = 2 * k
            start(c0 + 1, idx1_v, rows1_v, sem1)
            drain(c0, idx0_v, rows0_v, sem0)
            pltpu.sync_copy(rows0_v, acc_sh.at[idx0_v], add=True)
            start(c0 + 2, idx0_v, rows0_v, sem0)
            drain(c0 + 1, idx1_v, rows1_v, sem1)
            pltpu.sync_copy(rows1_v, acc_sh.at[idx1_v], add=True)

        drain(SNCH - 1, idx0_v, rows0_v, sem0)
        pltpu.sync_copy(rows0_v, acc_sh.at[idx0_v], add=True)

        plsc.subcore_barrier()

        @pl.loop(0, RPT // ZR)
        def _(j):
            r0 = sid * RPT + j * ZR
            pltpu.sync_copy(acc_sh.at[pl.ds(r0, ZR)], zbuf_v)
            pltpu.sync_copy(zbuf_v, out_hbm.at[cid, pl.ds(r0, ZR)])

    return sk(y, receivers)


def _node_body(*refs):
    nf_ref = refs[0]
    part_refs = refs[1:-8]
    (w0a_ref, w0b_ref, b0_ref, w1_ref, b1_ref, g_ref, beta_ref,
     out_ref) = refs[-8:]
    nf = nf_ref[...]
    acc = part_refs[0][...]
    for p in part_refs[1:]:
        acc = acc + p[...]
    x = _dot(nf, w0a_ref[...]) + _dot(acc, w0b_ref[...]) + b0_ref[...]
    h = jnp.maximum(x, 0.0)
    y = _dot(h, w1_ref[...]) + b1_ref[...]
    mu = jnp.mean(y, axis=1, keepdims=True)
    d = y - mu
    var = jnp.mean(d * d, axis=1, keepdims=True)
    out_ref[...] = d * lax.rsqrt(var + 1e-5) * g_ref[...] + beta_ref[...] + nf


def _node_mlp(nf, parts, w0a, w0b, b0, w1, b1, g, beta, bn=2000):
    grid = (N // bn,)
    row = lambda i: (i, 0)
    full = lambda i: (0, 0)
    return pl.pallas_call(
        _node_body,
        grid=grid,
        in_specs=(
            [pl.BlockSpec((bn, D), row)]
            + [pl.BlockSpec((bn, D), row)] * len(parts)
            + [
                pl.BlockSpec((D, D), full),
                pl.BlockSpec((D, D), full),
                pl.BlockSpec((1, D), full),
                pl.BlockSpec((D, D), full),
                pl.BlockSpec((1, D), full),
                pl.BlockSpec((1, D), full),
                pl.BlockSpec((1, D), full),
            ]
        ),
        out_specs=pl.BlockSpec((bn, D), row),
        out_shape=jax.ShapeDtypeStruct((N, D), jnp.float32),
    )(nf, *parts, w0a, w0b, b0, w1, b1, g, beta)


def kernel(senders, receivers, node_features, edge_features, params):
    nf = node_features.reshape(N, D)
    ef = edge_features.reshape(E, D)
    s = senders.reshape(E).astype(jnp.int32)
    r = receivers.reshape(E).astype(jnp.int32)

    pe = params["edge"]
    pn = params["node"]
    w0 = pe["W0"]
    w0s, w0r, w0e = w0[:D], w0[D:2 * D], w0[2 * D:]
    row = lambda v: v.reshape(1, D)

    ps, pr = _project(nf, w0s, w0r)

    chain = _alloc_edge_buf()
    parts = []
    for q in range(NQ):
        sl = slice(q * EQ, (q + 1) * EQ)
        gs, gr = _sc_gather(ps, pr, s[sl], r[sl])
        y, chain = _edge_mlp(gs, gr, ef, chain, q, w0e, row(pe["b0"]),
                             pe["W1"], row(pe["b1"]), row(pe["g"]),
                             row(pe["beta"]))
        pq = _sc_scatter(y, r[sl])
        parts.extend([pq[0, :N], pq[1, :N]])
    new_edge = chain
    w0n = pn["W0"]
    new_node = _node_mlp(nf, parts, w0n[:D], w0n[D:],
                         row(pn["b0"]), pn["W1"], row(pn["b1"]),
                         row(pn["g"]), row(pn["beta"]))

    return (new_node.reshape(1, N, D), new_edge.reshape(1, E, D))


# speedup vs baseline: 1.1509x; 1.1509x over previous
import functools

import jax
import jax.numpy as jnp
from jax import lax
from jax.experimental import pallas as pl
from jax.experimental.pallas import tpu as pltpu
from jax.experimental.pallas import tpu_sc as plsc

N = 10000
E = 320000
D = 128

NC = 2
NS = 16
NW = NC * NS
GW = 128
EPW = E // NW
SCH = 80
SNCH = EPW // SCH
NP = 10240
RPT = NP // NS
ZR = 128

_PREC = lax.Precision.DEFAULT


def _dot(a, b):
    return lax.dot_general(a, b, (((1,), (0,)), ((), ())), precision=_PREC,
                           preferred_element_type=jnp.float32)


def _project_body(nf_ref, w0s_ref, w0r_ref, ps_ref, pr_ref):
    nf = nf_ref[...]
    ps_ref[...] = _dot(nf, w0s_ref[...])
    pr_ref[...] = _dot(nf, w0r_ref[...])


def _project(nf, w0s, w0r, bn=2000):
    grid = (N // bn,)
    return pl.pallas_call(
        _project_body,
        grid=grid,
        in_specs=[
            pl.BlockSpec((bn, D), lambda i: (i, 0)),
            pl.BlockSpec((D, D), lambda i: (0, 0)),
            pl.BlockSpec((D, D), lambda i: (0, 0)),
        ],
        out_specs=[
            pl.BlockSpec((bn, D), lambda i: (i, 0)),
            pl.BlockSpec((bn, D), lambda i: (i, 0)),
        ],
        out_shape=[
            jax.ShapeDtypeStruct((N, D), jnp.float32),
            jax.ShapeDtypeStruct((N, D), jnp.float32),
        ],
    )(nf, w0s, w0r)


def _sc_gather(ps, pr, senders, receivers):
    ne = senders.shape[0]
    mesh = plsc.VectorSubcoreMesh(core_axis_name="core",
                                  subcore_axis_name="subcore")

    @functools.partial(
        pl.kernel,
        out_type=(
            jax.ShapeDtypeStruct((ne, D), jnp.float32),
            jax.ShapeDtypeStruct((ne, D), jnp.float32),
        ),
        mesh=mesh,
        scratch_types=[
            pltpu.SemaphoreType.DMA,
            pltpu.SemaphoreType.DMA,
        ],
    )
    def gk(ps_hbm, pr_hbm, s_hbm, r_hbm, gs_hbm, gr_hbm, sem_s, sem_r):
        def body(si_v, ri_v, gs_v, gr_v):
            cs = pltpu.make_async_copy(ps_hbm.at[si_v.at[0]], gs_v, sem_s)
            cr = pltpu.make_async_copy(pr_hbm.at[ri_v.at[0]], gr_v, sem_r)
            cs.start()
            cr.start()
            cs.wait()
            cr.wait()

        pltpu.emit_pipeline(
            body,
            grid=(ne // GW,),
            in_specs=[
                pl.BlockSpec((1, GW), lambda i: (0, i)),
                pl.BlockSpec((1, GW), lambda i: (0, i)),
            ],
            out_specs=[
                pl.BlockSpec((GW, D), lambda i: (i, 0)),
                pl.BlockSpec((GW, D), lambda i: (i, 0)),
            ],
            core_axis_name=("core", "subcore"),
            dimension_semantics=(pltpu.PARALLEL,),
        )(s_hbm, r_hbm, gs_hbm, gr_hbm)

    return gk(ps, pr, senders.reshape(1, ne), receivers.reshape(1, ne))


def _edge_body(gs_ref, gr_ref, ef_ref, w0e_ref, b0_ref, w1_ref, b1_ref,
               g_ref, beta_ref, y_ref, out_ref):
    ef = ef_ref[...]
    x = gs_ref[...] + gr_ref[...] + _dot(ef, w0e_ref[...]) + b0_ref[...]
    h = jnp.maximum(x, 0.0)
    y = _dot(h, w1_ref[...]) + b1_ref[...]
    mu = jnp.mean(y, axis=1, keepdims=True)
    d = y - mu
    var = jnp.mean(d * d, axis=1, keepdims=True)
    yln = d * lax.rsqrt(var + 1e-5) * g_ref[...] + beta_ref[...]
    y_ref[...] = yln
    out_ref[...] = yln + ef


def _edge_mlp(gs, gr, ef, w0e, b0, w1, b1, g, beta, be=4000):
    ne = ef.shape[0]
    grid = (ne // be,)
    row = lambda i: (i, 0)
    full = lambda i: (0, 0)
    return pl.pallas_call(
        _edge_body,
        grid=grid,
        in_specs=[
            pl.BlockSpec((be, D), row),
            pl.BlockSpec((be, D), row),
            pl.BlockSpec((be, D), row),
            pl.BlockSpec((D, D), full),
            pl.BlockSpec((1, D), full),
            pl.BlockSpec((D, D), full),
            pl.BlockSpec((1, D), full),
            pl.BlockSpec((1, D), full),
            pl.BlockSpec((1, D), full),
        ],
        out_specs=[
            pl.BlockSpec((be, D), row),
            pl.BlockSpec((be, D), row),
        ],
        out_shape=[
            jax.ShapeDtypeStruct((ne, D), jnp.float32),
            jax.ShapeDtypeStruct((ne, D), jnp.float32),
        ],
    )(gs, gr, ef, w0e, b0, w1, b1, g, beta)


def _sc_scatter(y, receivers):
    mesh = plsc.VectorSubcoreMesh(core_axis_name="core",
                                  subcore_axis_name="subcore")

    @functools.partial(
        pl.kernel,
        out_type=jax.ShapeDtypeStruct((NC, NP, D), jnp.float32),
        mesh=mesh,
        scratch_types=[
            pltpu.VMEM((SCH,), jnp.int32),
            pltpu.VMEM((SCH,), jnp.int32),
            pltpu.VMEM((SCH, D), jnp.float32),
            pltpu.VMEM((SCH, D), jnp.float32),
            pltpu.VMEM((ZR, D), jnp.float32),
            pltpu.VMEM_SHARED((NP, D), jnp.float32),
            pltpu.SemaphoreType.DMA,
            pltpu.SemaphoreType.DMA,
        ],
    )
    def sk(y_hbm, r_hbm, out_hbm, idx0_v, idx1_v, rows0_v, rows1_v, zbuf_v,
           acc_sh, sem0, sem1):
        cid = lax.axis_index("core")
        sid = lax.axis_index("subcore")
        wid = cid * NS + sid

        def start(c, idx_v, rows_v, sem):
            base = wid * EPW + c * SCH
            ci = pltpu.make_async_copy(r_hbm.at[pl.ds(base, SCH)], idx_v, sem)
            cy = pltpu.make_async_copy(y_hbm.at[pl.ds(base, SCH)], rows_v, sem)
            ci.start()
            cy.start()
            return ci, cy

        def drain(c, idx_v, rows_v, sem):
            ci = pltpu.make_async_copy(r_hbm.at[pl.ds(0, SCH)], idx_v, sem)
            cy = pltpu.make_async_copy(y_hbm.at[pl.ds(0, SCH)], rows_v, sem)
            ci.wait()
            cy.wait()

        @pl.loop(0, ZR)
        def _(r):
            @pl.loop(0, D // 16)
            def _(c):
                zbuf_v[r, pl.ds(c * 16, 16)] = jnp.zeros((16,), jnp.float32)

        @pl.loop(0, RPT // ZR)
        def _(j):
            pltpu.sync_copy(zbuf_v, acc_sh.at[pl.ds(sid * RPT + j * ZR, ZR)])

        plsc.subcore_barrier()

        start(0, idx0_v, rows0_v, sem0)

        @pl.loop(0, (SNCH - 1) // 2)
        def _(k):
            c0 = 2 * k
            start(c0 + 1, idx1_v, rows1_v, sem1)
            drain(c0, idx0_v, rows0_v, sem0)
            pltpu.sync_copy(rows0_v, acc_sh.at[idx0_v], add=True)
            start(c0 + 2, idx0_v, rows0_v, sem0)
            drain(c0 + 1, idx1_v, rows1_v, sem1)
            pltpu.sync_copy(rows1_v, acc_sh.at[idx1_v], add=True)

        drain(SNCH - 1, idx0_v, rows0_v, sem0)
        pltpu.sync_copy(rows0_v, acc_sh.at[idx0_v], add=True)

        plsc.subcore_barrier()

        @pl.loop(0, RPT // ZR)
        def _(j):
            r0 = sid * RPT + j * ZR
            pltpu.sync_copy(acc_sh.at[pl.ds(r0, ZR)], zbuf_v)
            pltpu.sync_copy(zbuf_v, out_hbm.at[cid, pl.ds(r0, ZR)])

    return sk(y, receivers)


def _node_body(*refs):
    nf_ref = refs[0]
    part_refs = refs[1:-8]
    (w0a_ref, w0b_ref, b0_ref, w1_ref, b1_ref, g_ref, beta_ref,
     out_ref) = refs[-8:]
    nf = nf_ref[...]
    acc = part_refs[0][...]
    for p in part_refs[1:]:
        acc = acc + p[...]
    x = _dot(nf, w0a_ref[...]) + _dot(acc, w0b_ref[...]) + b0_ref[...]
    h = jnp.maximum(x, 0.0)
    y = _dot(h, w1_ref[...]) + b1_ref[...]
    mu = jnp.mean(y, axis=1, keepdims=True)
    d = y - mu
    var = jnp.mean(d * d, axis=1, keepdims=True)
    out_ref[...] = d * lax.rsqrt(var + 1e-5) * g_ref[...] + beta_ref[...] + nf


def _node_mlp(nf, parts, w0a, w0b, b0, w1, b1, g, beta, bn=2000):
    grid = (N // bn,)
    row = lambda i: (i, 0)
    full = lambda i: (0, 0)
    return pl.pallas_call(
        _node_body,
        grid=grid,
        in_specs=(
            [pl.BlockSpec((bn, D), row)]
            + [pl.BlockSpec((bn, D), row)] * len(parts)
            + [
                pl.BlockSpec((D, D), full),
                pl.BlockSpec((D, D), full),
                pl.BlockSpec((1, D), full),
                pl.BlockSpec((D, D), full),
                pl.BlockSpec((1, D), full),
                pl.BlockSpec((1, D), full),
                pl.BlockSpec((1, D), full),
            ]
        ),
        out_specs=pl.BlockSpec((bn, D), row),
        out_shape=jax.ShapeDtypeStruct((N, D), jnp.float32),
    )(nf, *parts, w0a, w0b, b0, w1, b1, g, beta)


def kernel(senders, receivers, node_features, edge_features, params):
    nf = node_features.reshape(N, D)
    ef = edge_features.reshape(E, D)
    s = senders.reshape(E).astype(jnp.int32)
    r = receivers.reshape(E).astype(jnp.int32)

    pe = params["edge"]
    pn = params["node"]
    w0 = pe["W0"]
    w0s, w0r, w0e = w0[:D], w0[D:2 * D], w0[2 * D:]
    row = lambda v: v.reshape(1, D)

    ps, pr = _project(nf, w0s, w0r)
    gs, gr = _sc_gather(ps, pr, s, r)
    y, new_edge = _edge_mlp(gs, gr, ef, w0e, row(pe["b0"]), pe["W1"],
                            row(pe["b1"]), row(pe["g"]), row(pe["beta"]))
    partials = _sc_scatter(y, r)
    parts = [partials[0, :N], partials[1, :N]]
    w0n = pn["W0"]
    new_node = _node_mlp(nf, parts, w0n[:D], w0n[D:],
                         row(pn["b0"]), pn["W1"], row(pn["b1"]),
                         row(pn["g"]), row(pn["beta"]))

    return (new_node.reshape(1, N, D), new_edge.reshape(1, E, D))


# speedup vs baseline: 1.1667x; 1.0137x over previous
import functools

import jax
import jax.numpy as jnp
from jax import lax
from jax.experimental import pallas as pl
from jax.experimental.pallas import tpu as pltpu
from jax.experimental.pallas import tpu_sc as plsc

N = 10000
E = 320000
D = 128

NC = 2
NS = 16
NW = NC * NS
GW = 128
EPW = E // NW
SCH = 80
SNCH = EPW // SCH
NP = 10240
RPT = NP // NS
ZR = 128

_PREC = lax.Precision.DEFAULT


def _dot(a, b):
    return lax.dot_general(a, b, (((1,), (0,)), ((), ())), precision=_PREC,
                           preferred_element_type=jnp.float32)


def _project_body(nf_ref, w0s_ref, w0r_ref, ps_ref, pr_ref):
    nf = nf_ref[...]
    ps_ref[...] = _dot(nf, w0s_ref[...])
    pr_ref[...] = _dot(nf, w0r_ref[...])


def _project(nf, w0s, w0r, bn=2000):
    grid = (N // bn,)
    return pl.pallas_call(
        _project_body,
        grid=grid,
        in_specs=[
            pl.BlockSpec((bn, D), lambda i: (i, 0)),
            pl.BlockSpec((D, D), lambda i: (0, 0)),
            pl.BlockSpec((D, D), lambda i: (0, 0)),
        ],
        out_specs=[
            pl.BlockSpec((bn, D), lambda i: (i, 0)),
            pl.BlockSpec((bn, D), lambda i: (i, 0)),
        ],
        out_shape=[
            jax.ShapeDtypeStruct((N, D), jnp.float32),
            jax.ShapeDtypeStruct((N, D), jnp.float32),
        ],
    )(nf, w0s, w0r)


def _sc_gather(ps, pr, senders, receivers):
    ne = senders.shape[0]
    mesh = plsc.VectorSubcoreMesh(core_axis_name="core",
                                  subcore_axis_name="subcore")

    @functools.partial(
        pl.kernel,
        out_type=(
            jax.ShapeDtypeStruct((ne, D), jnp.float32),
            jax.ShapeDtypeStruct((ne, D), jnp.float32),
        ),
        mesh=mesh,
        scratch_types=[
            pltpu.SemaphoreType.DMA,
            pltpu.SemaphoreType.DMA,
        ],
    )
    def gk(ps_hbm, pr_hbm, s_hbm, r_hbm, gs_hbm, gr_hbm, sem_s, sem_r):
        def body(si_v, ri_v, gs_v, gr_v):
            cs = pltpu.make_async_copy(ps_hbm.at[si_v.at[0]], gs_v, sem_s)
            cr = pltpu.make_async_copy(pr_hbm.at[ri_v.at[0]], gr_v, sem_r)
            cs.start()
            cr.start()
            cs.wait()
            cr.wait()

        pltpu.emit_pipeline(
            body,
            grid=(ne // GW,),
            in_specs=[
                pl.BlockSpec((1, GW), lambda i: (0, i)),
                pl.BlockSpec((1, GW), lambda i: (0, i)),
            ],
            out_specs=[
                pl.BlockSpec((GW, D), lambda i: (i, 0)),
                pl.BlockSpec((GW, D), lambda i: (i, 0)),
            ],
            core_axis_name=("core", "subcore"),
            dimension_semantics=(pltpu.PARALLEL,),
        )(s_hbm, r_hbm, gs_hbm, gr_hbm)

    return gk(ps, pr, senders.reshape(1, ne), receivers.reshape(1, ne))


def _edge_body(gs_ref, gr_ref, ef_ref, w0e_ref, b0_ref, w1_ref, b1_ref,
               g_ref, beta_ref, y_ref, out_ref):
    ef = ef_ref[...]
    x = gs_ref[...] + gr_ref[...] + _dot(ef, w0e_ref[...]) + b0_ref[...]
    h = jnp.maximum(x, 0.0)
    y = _dot(h, w1_ref[...]) + b1_ref[...]
    mu = jnp.mean(y, axis=1, keepdims=True)
    d = y - mu
    var = jnp.mean(d * d, axis=1, keepdims=True)
    yln = d * lax.rsqrt(var + 1e-5) * g_ref[...] + beta_ref[...]
    y_ref[...] = yln
    out_ref[...] = yln + ef


def _edge_mlp(gs, gr, ef, w0e, b0, w1, b1, g, beta, be=8000):
    ne = ef.shape[0]
    grid = (ne // be,)
    row = lambda i: (i, 0)
    full = lambda i: (0, 0)
    return pl.pallas_call(
        _edge_body,
        grid=grid,
        in_specs=[
            pl.BlockSpec((be, D), row),
            pl.BlockSpec((be, D), row),
            pl.BlockSpec((be, D), row),
            pl.BlockSpec((D, D), full),
            pl.BlockSpec((1, D), full),
            pl.BlockSpec((D, D), full),
            pl.BlockSpec((1, D), full),
            pl.BlockSpec((1, D), full),
            pl.BlockSpec((1, D), full),
        ],
        out_specs=[
            pl.BlockSpec((be, D), row),
            pl.BlockSpec((be, D), row),
        ],
        out_shape=[
            jax.ShapeDtypeStruct((ne, D), jnp.float32),
            jax.ShapeDtypeStruct((ne, D), jnp.float32),
        ],
    )(gs, gr, ef, w0e, b0, w1, b1, g, beta)


def _sc_scatter(y, receivers):
    mesh = plsc.VectorSubcoreMesh(core_axis_name="core",
                                  subcore_axis_name="subcore")

    @functools.partial(
        pl.kernel,
        out_type=jax.ShapeDtypeStruct((NC, NP, D), jnp.float32),
        mesh=mesh,
        scratch_types=[
            pltpu.VMEM((SCH,), jnp.int32),
            pltpu.VMEM((SCH,), jnp.int32),
            pltpu.VMEM((SCH, D), jnp.float32),
            pltpu.VMEM((SCH, D), jnp.float32),
            pltpu.VMEM((ZR, D), jnp.float32),
            pltpu.VMEM_SHARED((NP, D), jnp.float32),
            pltpu.SemaphoreType.DMA,
            pltpu.SemaphoreType.DMA,
        ],
    )
    def sk(y_hbm, r_hbm, out_hbm, idx0_v, idx1_v, rows0_v, rows1_v, zbuf_v,
           acc_sh, sem0, sem1):
        cid = lax.axis_index("core")
        sid = lax.axis_index("subcore")
        wid = cid * NS + sid

        def start(c, idx_v, rows_v, sem):
            base = wid * EPW + c * SCH
            ci = pltpu.make_async_copy(r_hbm.at[pl.ds(base, SCH)], idx_v, sem)
            cy = pltpu.make_async_copy(y_hbm.at[pl.ds(base, SCH)], rows_v, sem)
            ci.start()
            cy.start()
            return ci, cy

        def drain(c, idx_v, rows_v, sem):
            ci = pltpu.make_async_copy(r_hbm.at[pl.ds(0, SCH)], idx_v, sem)
            cy = pltpu.make_async_copy(y_hbm.at[pl.ds(0, SCH)], rows_v, sem)
            ci.wait()
            cy.wait()

        @pl.loop(0, ZR)
        def _(r):
            @pl.loop(0, D // 16)
            def _(c):
                zbuf_v[r, pl.ds(c * 16, 16)] = jnp.zeros((16,), jnp.float32)

        @pl.loop(0, RPT // ZR)
        def _(j):
            pltpu.sync_copy(zbuf_v, acc_sh.at[pl.ds(sid * RPT + j * ZR, ZR)])

        plsc.subcore_barrier()

        start(0, idx0_v, rows0_v, sem0)

        @pl.loop(0, (SNCH - 1) // 2)
        def _(k):
            c0 = 2 * k
            start(c0 + 1, idx1_v, rows1_v, sem1)
            drain(c0, idx0_v, rows0_v, sem0)
            pltpu.sync_copy(rows0_v, acc_sh.at[idx0_v], add=True)
            start(c0 + 2, idx0_v, rows0_v, sem0)
            drain(c0 + 1, idx1_v, rows1_v, sem1)
            pltpu.sync_copy(rows1_v, acc_sh.at[idx1_v], add=True)

        drain(SNCH - 1, idx0_v, rows0_v, sem0)
        pltpu.sync_copy(rows0_v, acc_sh.at[idx0_v], add=True)

        plsc.subcore_barrier()

        @pl.loop(0, RPT // ZR)
        def _(j):
            r0 = sid * RPT + j * ZR
            pltpu.sync_copy(acc_sh.at[pl.ds(r0, ZR)], zbuf_v)
            pltpu.sync_copy(zbuf_v, out_hbm.at[cid, pl.ds(r0, ZR)])

    return sk(y, receivers)


def _node_body(*refs):
    nf_ref = refs[0]
    part_refs = refs[1:-8]
    (w0a_ref, w0b_ref, b0_ref, w1_ref, b1_ref, g_ref, beta_ref,
     out_ref) = refs[-8:]
    nf = nf_ref[...]
    acc = part_refs[0][...]
    for p in part_refs[1:]:
        acc = acc + p[...]
    x = _dot(nf, w0a_ref[...]) + _dot(acc, w0b_ref[...]) + b0_ref[...]
    h = jnp.maximum(x, 0.0)
    y = _dot(h, w1_ref[...]) + b1_ref[...]
    mu = jnp.mean(y, axis=1, keepdims=True)
    d = y - mu
    var = jnp.mean(d * d, axis=1, keepdims=True)
    out_ref[...] = d * lax.rsqrt(var + 1e-5) * g_ref[...] + beta_ref[...] + nf


def _node_mlp(nf, parts, w0a, w0b, b0, w1, b1, g, beta, bn=2000):
    grid = (N // bn,)
    row = lambda i: (i, 0)
    full = lambda i: (0, 0)
    return pl.pallas_call(
        _node_body,
        grid=grid,
        in_specs=(
            [pl.BlockSpec((bn, D), row)]
            + [pl.BlockSpec((bn, D), row)] * len(parts)
            + [
                pl.BlockSpec((D, D), full),
                pl.BlockSpec((D, D), full),
                pl.BlockSpec((1, D), full),
                pl.BlockSpec((D, D), full),
                pl.BlockSpec((1, D), full),
                pl.BlockSpec((1, D), full),
                pl.BlockSpec((1, D), full),
            ]
        ),
        out_specs=pl.BlockSpec((bn, D), row),
        out_shape=jax.ShapeDtypeStruct((N, D), jnp.float32),
    )(nf, *parts, w0a, w0b, b0, w1, b1, g, beta)


def kernel(senders, receivers, node_features, edge_features, params):
    nf = node_features.reshape(N, D)
    ef = edge_features.reshape(E, D)
    s = senders.reshape(E).astype(jnp.int32)
    r = receivers.reshape(E).astype(jnp.int32)

    pe = params["edge"]
    pn = params["node"]
    w0 = pe["W0"]
    w0s, w0r, w0e = w0[:D], w0[D:2 * D], w0[2 * D:]
    row = lambda v: v.reshape(1, D)

    ps, pr = _project(nf, w0s, w0r)
    gs, gr = _sc_gather(ps, pr, s, r)
    y, new_edge = _edge_mlp(gs, gr, ef, w0e, row(pe["b0"]), pe["W1"],
                            row(pe["b1"]), row(pe["g"]), row(pe["beta"]))
    partials = _sc_scatter(y, r)
    parts = [partials[0, :N], partials[1, :N]]
    w0n = pn["W0"]
    new_node = _node_mlp(nf, parts, w0n[:D], w0n[D:],
                         row(pn["b0"]), pn["W1"], row(pn["b1"]),
                         row(pn["g"]), row(pn["beta"]))

    return (new_node.reshape(1, N, D), new_edge.reshape(1, E, D))


# speedup vs baseline: 1.1866x; 1.0171x over previous
import functools

import jax
import jax.numpy as jnp
from jax import lax
from jax.experimental import pallas as pl
from jax.experimental.pallas import tpu as pltpu
from jax.experimental.pallas import tpu_sc as plsc

N = 10000
E = 320000
D = 128

NC = 2
NS = 16
NW = NC * NS
GW = 128
SCH = 128
NP = 10240
RPT = NP // NS
ZR = 64

_PREC = lax.Precision.DEFAULT


def _dot(a, b):
    return lax.dot_general(a, b, (((1,), (0,)), ((), ())), precision=_PREC,
                           preferred_element_type=jnp.float32)


def _project_body(nf_ref, w0s_ref, w0r_ref, ps_ref, pr_ref):
    nf = nf_ref[...]
    ps_ref[...] = _dot(nf, w0s_ref[...])
    pr_ref[...] = _dot(nf, w0r_ref[...])


def _project(nf, w0s, w0r, bn=2000):
    grid = (N // bn,)
    return pl.pallas_call(
        _project_body,
        grid=grid,
        in_specs=[
            pl.BlockSpec((bn, D), lambda i: (i, 0)),
            pl.BlockSpec((D, D), lambda i: (0, 0)),
            pl.BlockSpec((D, D), lambda i: (0, 0)),
        ],
        out_specs=[
            pl.BlockSpec((bn, D), lambda i: (i, 0)),
            pl.BlockSpec((bn, D), lambda i: (i, 0)),
        ],
        out_shape=[
            jax.ShapeDtypeStruct((N, D), jnp.float32),
            jax.ShapeDtypeStruct((N, D), jnp.float32),
        ],
    )(nf, w0s, w0r)


def _sc_gather(ps, pr, senders, receivers):
    ne = senders.shape[0]
    mesh = plsc.VectorSubcoreMesh(core_axis_name="core",
                                  subcore_axis_name="subcore")

    @functools.partial(
        pl.kernel,
        out_type=(
            jax.ShapeDtypeStruct((ne, D), jnp.float32),
            jax.ShapeDtypeStruct((ne, D), jnp.float32),
        ),
        mesh=mesh,
        scratch_types=[
            pltpu.SemaphoreType.DMA,
            pltpu.SemaphoreType.DMA,
        ],
    )
    def gk(ps_hbm, pr_hbm, s_hbm, r_hbm, gs_hbm, gr_hbm, sem_s, sem_r):
        def body(si_v, ri_v, gs_v, gr_v):
            cs = pltpu.make_async_copy(ps_hbm.at[si_v.at[0]], gs_v, sem_s)
            cr = pltpu.make_async_copy(pr_hbm.at[ri_v.at[0]], gr_v, sem_r)
            cs.start()
            cr.start()
            cs.wait()
            cr.wait()

        pltpu.emit_pipeline(
            body,
            grid=(ne // GW,),
            in_specs=[
                pl.BlockSpec((1, GW), lambda i: (0, i)),
                pl.BlockSpec((1, GW), lambda i: (0, i)),
            ],
            out_specs=[
                pl.BlockSpec((GW, D), lambda i: (i, 0)),
                pl.BlockSpec((GW, D), lambda i: (i, 0)),
            ],
            core_axis_name=("core", "subcore"),
            dimension_semantics=(pltpu.PARALLEL,),
        )(s_hbm, r_hbm, gs_hbm, gr_hbm)

    return gk(ps, pr, senders.reshape(1, ne), receivers.reshape(1, ne))


def _edge_body(gs_ref, gr_ref, ef_ref, w0e_ref, b0_ref, w1_ref, b1_ref,
               g_ref, beta_ref, y_ref, out_ref):
    ef = ef_ref[...]
    x = gs_ref[...] + gr_ref[...] + _dot(ef, w0e_ref[...]) + b0_ref[...]
    h = jnp.maximum(x, 0.0)
    y = _dot(h, w1_ref[...]) + b1_ref[...]
    mu = jnp.mean(y, axis=1, keepdims=True)
    d = y - mu
    var = jnp.mean(d * d, axis=1, keepdims=True)
    yln = d * lax.rsqrt(var + 1e-5) * g_ref[...] + beta_ref[...]
    y_ref[...] = yln
    out_ref[...] = yln + ef


def _edge_mlp(gs, gr, ef, w0e, b0, w1, b1, g, beta, be=8000):
    ne = ef.shape[0]
    grid = (ne // be,)
    row = lambda i: (i, 0)
    full = lambda i: (0, 0)
    return pl.pallas_call(
        _edge_body,
        grid=grid,
        in_specs=[
            pl.BlockSpec((be, D), row),
            pl.BlockSpec((be, D), row),
            pl.BlockSpec((be, D), row),
            pl.BlockSpec((D, D), full),
            pl.BlockSpec((1, D), full),
            pl.BlockSpec((D, D), full),
            pl.BlockSpec((1, D), full),
            pl.BlockSpec((1, D), full),
            pl.BlockSpec((1, D), full),
        ],
        out_specs=[
            pl.BlockSpec((be, D), row),
            pl.BlockSpec((be, D), row),
        ],
        out_shape=[
            jax.ShapeDtypeStruct((ne, D), jnp.float32),
            jax.ShapeDtypeStruct((ne, D), jnp.float32),
        ],
    )(gs, gr, ef, w0e, b0, w1, b1, g, beta)


def _sc_scatter(y, receivers):
    mesh = plsc.VectorSubcoreMesh(core_axis_name="core",
                                  subcore_axis_name="subcore")
    nwin = y.shape[0] // SCH
    kmax = (nwin + NW - 1) // NW

    @functools.partial(
        pl.kernel,
        out_type=jax.ShapeDtypeStruct((NC, NP, D), jnp.float32),
        mesh=mesh,
        scratch_types=[
            pltpu.VMEM((SCH,), jnp.int32),
            pltpu.VMEM((SCH,), jnp.int32),
            pltpu.VMEM((SCH, D), jnp.float32),
            pltpu.VMEM((SCH, D), jnp.float32),
            pltpu.VMEM((ZR, D), jnp.float32),
            pltpu.VMEM_SHARED((NP, D), jnp.float32),
            pltpu.SemaphoreType.DMA,
            pltpu.SemaphoreType.DMA,
        ],
    )
    def sk(y_hbm, r_hbm, out_hbm, idx0_v, idx1_v, rows0_v, rows1_v, zbuf_v,
           acc_sh, sem0, sem1):
        cid = lax.axis_index("core")
        sid = lax.axis_index("subcore")
        wid = cid * NS + sid

        def start(k, idx_v, rows_v, sem):
            widx = k * NW + wid

            @pl.when(widx < nwin)
            def _():
                base = widx * SCH
                pltpu.make_async_copy(
                    r_hbm.at[pl.ds(base, SCH)], idx_v, sem).start()
                pltpu.make_async_copy(
                    y_hbm.at[pl.ds(base, SCH)], rows_v, sem).start()

        def drain_add(k, idx_v, rows_v, sem):
            widx = k * NW + wid

            @pl.when(widx < nwin)
            def _():
                pltpu.make_async_copy(
                    r_hbm.at[pl.ds(0, SCH)], idx_v, sem).wait()
                pltpu.make_async_copy(
                    y_hbm.at[pl.ds(0, SCH)], rows_v, sem).wait()
                pltpu.sync_copy(rows_v, acc_sh.at[idx_v], add=True)

        @pl.loop(0, ZR)
        def _(r):
            @pl.loop(0, D // 16)
            def _(c):
                zbuf_v[r, pl.ds(c * 16, 16)] = jnp.zeros((16,), jnp.float32)

        @pl.loop(0, RPT // ZR)
        def _(j):
            pltpu.sync_copy(zbuf_v, acc_sh.at[pl.ds(sid * RPT + j * ZR, ZR)])

        plsc.subcore_barrier()

        start(0, idx0_v, rows0_v, sem0)

        @pl.loop(0, (kmax - 1) // 2)
        def _(j):
            k0 = 2 * j
            start(k0 + 1, idx1_v, rows1_v, sem1)
            drain_add(k0, idx0_v, rows0_v, sem0)
            start(k0 + 2, idx0_v, rows0_v, sem0)
            drain_add(k0 + 1, idx1_v, rows1_v, sem1)

        drain_add(kmax - 1, idx0_v, rows0_v, sem0)

        plsc.subcore_barrier()

        @pl.loop(0, RPT // ZR)
        def _(j):
            r0 = sid * RPT + j * ZR
            pltpu.sync_copy(acc_sh.at[pl.ds(r0, ZR)], zbuf_v)
            pltpu.sync_copy(zbuf_v, out_hbm.at[cid, pl.ds(r0, ZR)])

    return sk(y, receivers)


def _node_body(*refs):
    nf_ref = refs[0]
    part_refs = refs[1:-8]
    (w0a_ref, w0b_ref, b0_ref, w1_ref, b1_ref, g_ref, beta_ref,
     out_ref) = refs[-8:]
    nf = nf_ref[...]
    acc = part_refs[0][...]
    for p in part_refs[1:]:
        acc = acc + p[...]
    x = _dot(nf, w0a_ref[...]) + _dot(acc, w0b_ref[...]) + b0_ref[...]
    h = jnp.maximum(x, 0.0)
    y = _dot(h, w1_ref[...]) + b1_ref[...]
    mu = jnp.mean(y, axis=1, keepdims=True)
    d = y - mu
    var = jnp.mean(d * d, axis=1, keepdims=True)
    out_ref[...] = d * lax.rsqrt(var + 1e-5) * g_ref[...] + beta_ref[...] + nf


def _node_mlp(nf, parts, w0a, w0b, b0, w1, b1, g, beta, bn=2000):
    grid = (N // bn,)
    row = lambda i: (i, 0)
    full = lambda i: (0, 0)
    return pl.pallas_call(
        _node_body,
        grid=grid,
        in_specs=(
            [pl.BlockSpec((bn, D), row)]
            + [pl.BlockSpec((bn, D), row)] * len(parts)
            + [
                pl.BlockSpec((D, D), full),
                pl.BlockSpec((D, D), full),
                pl.BlockSpec((1, D), full),
                pl.BlockSpec((D, D), full),
                pl.BlockSpec((1, D), full),
                pl.BlockSpec((1, D), full),
                pl.BlockSpec((1, D), full),
            ]
        ),
        out_specs=pl.BlockSpec((bn, D), row),
        out_shape=jax.ShapeDtypeStruct((N, D), jnp.float32),
    )(nf, *parts, w0a, w0b, b0, w1, b1, g, beta)


def kernel(senders, receivers, node_features, edge_features, params):
    nf = node_features.reshape(N, D)
    ef = edge_features.reshape(E, D)
    s = senders.reshape(E).astype(jnp.int32)
    r = receivers.reshape(E).astype(jnp.int32)

    pe = params["edge"]
    pn = params["node"]
    w0 = pe["W0"]
    w0s, w0r, w0e = w0[:D], w0[D:2 * D], w0[2 * D:]
    row = lambda v: v.reshape(1, D)

    ps, pr = _project(nf, w0s, w0r)
    gs, gr = _sc_gather(ps, pr, s, r)
    y, new_edge = _edge_mlp(gs, gr, ef, w0e, row(pe["b0"]), pe["W1"],
                            row(pe["b1"]), row(pe["g"]), row(pe["beta"]))
    partials = _sc_scatter(y, r)
    parts = [partials[0, :N], partials[1, :N]]
    w0n = pn["W0"]
    new_node = _node_mlp(nf, parts, w0n[:D], w0n[D:],
                         row(pn["b0"]), pn["W1"], row(pn["b1"]),
                         row(pn["g"]), row(pn["beta"]))

    return (new_node.reshape(1, N, D), new_edge.reshape(1, E, D))
